# Initial kernel scaffold; baseline (speedup 1.0000x reference)
import jax
import jax.numpy as jnp
from jax.experimental import pallas as pl


def kernel(pm25, feature, edge_attr, wind_mean, wind_std, W_ih, W_hh, b_ih, b_hh, W1, b1, W2, b2, Wn, bn, Wn2, bn2, edge_index):
    raise NotImplementedError("write your pallas kernel here")


# speedup vs baseline: 3.0084x; 3.0084x over previous
import jax
import jax.numpy as jnp
from jax import lax
from jax.experimental import pallas as pl
from jax.experimental.pallas import tpu as pltpu
from jax.experimental.pallas import tpu_sc as plsc

B, N, E = 8, 500, 8000
H = 64
FD, HIST, PRED = 8, 1, 6
NP = B * PRED
NPAD = 512
EH = E // 2
NW = 32
RCHUNK = 125
NTASK = NP * (N // RCHUNK)
TPW = NTASK // NW
RB = RCHUNK * NPAD
NEG = -3.0e38
GBLK = 2000


def _prep_body(eaT_ref, w12_ref, bih_ref, featw_ref, nodein_ref,
               wn_ref, bn_ref, wn2_ref, bn2_ref, wm_ref, ws_ref,
               base_gx_ref, c1_ref, c2_ref, a_ref, b_ref, hn_ref):
    eaT = eaT_ref[...]
    mean = jnp.mean(eaT, axis=1, keepdims=True)
    var = jnp.sum((eaT - mean) ** 2, axis=1, keepdims=True) / (E - 1)
    ean_T = (eaT - mean) / jnp.sqrt(var)
    base_gx_ref[...] = lax.dot_general(
        ean_T, w12_ref[...], (((0,), (0,)), ((), ())),
        preferred_element_type=jnp.float32) + bih_ref[...]
    dist = eaT[0:1, :]
    direc = eaT[1:2, :]
    c1_ref[...] = 3.0 * jnp.cos(direc) / dist
    c2_ref[...] = 3.0 * jnp.sin(direc) / dist

    fw = featw_ref[...]
    speed = fw[0:1, :] * ws_ref[0] + wm_ref[0]
    wdir = fw[1:2, :] * ws_ref[1] + wm_ref[1]
    a_ref[...] = speed * jnp.cos(wdir)
    b_ref[...] = speed * jnp.sin(wdir)

    node_in = nodein_ref[...]
    h1 = jnp.tanh(lax.dot_general(
        wn_ref[...], node_in, (((0,), (0,)), ((), ())),
        preferred_element_type=jnp.float32) + bn_ref[...])
    hn = lax.dot_general(wn2_ref[...], h1, (((0,), (0,)), ((), ())),
                         preferred_element_type=jnp.float32)
    hn_ref[...] = hn + bn2_ref[0]


def _gru_body(base_gx_ref, ew_ref, whh_ref, bhh_ref, w2row_ref,
              w1_ref, b1_ref, w2_ref, b2_ref, out_ref):
    base_gx = base_gx_ref[...]
    ewb = ew_ref[0]
    en = jnp.zeros((GBLK, H), jnp.float32)
    for i in range(PRED):
        gx = base_gx + lax.dot_general(
            ewb[i:i + 1, :], w2row_ref[...], (((0,), (0,)), ((), ())),
            preferred_element_type=jnp.float32)
        gh = jnp.dot(en, whh_ref[...],
                     preferred_element_type=jnp.float32) + bhh_ref[...]
        r = jax.nn.sigmoid(gx[:, :H] + gh[:, :H])
        z = jax.nn.sigmoid(gx[:, H:2 * H] + gh[:, H:2 * H])
        n = jnp.tanh(gx[:, 2 * H:] + r * gh[:, 2 * H:])
        en = (1.0 - z) * n + z * en
        hid = jax.nn.relu(
            jnp.dot(en, w1_ref[...], preferred_element_type=jnp.float32)
            + b1_ref[...])
        rep = lax.dot_general(w2_ref[...], hid, (((0,), (1,)), ((), ())),
                              preferred_element_type=jnp.float32) + b2_ref[...]
        out_ref[0, i, :] = rep[0]


def _div6(p):
    return (p * 10923) >> 16


def _vrcp(x):
    xi = plsc.bitcast(x, jnp.int32)
    y = plsc.bitcast(jnp.int32(0x7EF127EA) - xi, jnp.float32)
    for _ in range(3):
        y = y * (2.0 - x * y)
    return y


def _ew_body(c1_hbm, c2_hbm, a_hbm, b_hbm, src_hbm, out_hbm,
             a_v, b_v, src_v, c1_v, c2_v, ew_v):
    wid = lax.axis_index("s") * 2 + lax.axis_index("c")
    for k in range(3):
        t = wid * 3 + k
        p = t >> 1
        h = t & 1
        bidx = _div6(p)
        i = p - PRED * bidx
        pltpu.sync_copy(a_hbm.at[pl.ds(p * NPAD, NPAD)], a_v)
        pltpu.sync_copy(b_hbm.at[pl.ds(p * NPAD, NPAD)], b_v)
        pltpu.sync_copy(src_hbm.at[pl.ds(h * EH, EH)], src_v)
        pltpu.sync_copy(c1_hbm.at[pl.ds(h * EH, EH)], c1_v)
        pltpu.sync_copy(c2_hbm.at[pl.ds(h * EH, EH)], c2_v)

        def body(tt, _):
            sl = pl.ds(tt * 16, 16)
            idx = src_v[sl]
            av = plsc.load_gather(a_v, [idx])
            bv = plsc.load_gather(b_v, [idx])
            ew_v[sl] = jnp.maximum(c1_v[sl] * av + c2_v[sl] * bv, 0.0)
            return 0

        lax.fori_loop(0, EH // 16, body, 0, unroll=4)
        kb = bidx * 4 + h * 2
        pltpu.sync_copy(ew_v.at[pl.ds(0, GBLK)],
                        out_hbm.at[pl.ds((kb * PRED + i) * GBLK, GBLK)])
        pltpu.sync_copy(ew_v.at[pl.ds(GBLK, GBLK)],
                        out_hbm.at[pl.ds(((kb + 1) * PRED + i) * GBLK, GBLK)])


def _scatter_body(v_hbm, hn_hbm, perm_hbm, lflat_hbm, off_hbm, zeros_hbm,
                  r_hbm, cp_hbm,
                  perm_v, lflat_v, off_v, v_v, hn_v, rbuf, cp_v):
    wid = lax.axis_index("s") * 2 + lax.axis_index("c")
    pltpu.sync_copy(perm_hbm, perm_v)
    pltpu.sync_copy(lflat_hbm, lflat_v)
    pltpu.sync_copy(off_hbm, off_v)
    lane = lax.broadcasted_iota(jnp.int32, (16,), 0)
    tail_mask = lane < (N - 31 * 16)

    def off_at(idx):
        g = plsc.load_gather(off_v, [jnp.full((16,), idx, jnp.int32)])
        return lax.reduce_max(g, axes=(0,))

    for k in range(TPW):
        t = wid * TPW + k
        p = t >> 2
        q = t & 3
        bidx = _div6(p)
        i = p - PRED * bidx
        r0 = q * RCHUNK
        for j in range(4):
            pltpu.sync_copy(
                v_hbm.at[pl.ds(((bidx * 4 + j) * PRED + i) * GBLK, GBLK)],
                v_v.at[pl.ds(j * GBLK, GBLK)])
        pltpu.sync_copy(hn_hbm.at[pl.ds(p * NPAD, NPAD)], hn_v)
        pltpu.sync_copy(zeros_hbm, rbuf)

        start = off_at(r0)
        end = off_at(r0 + RCHUNK)
        nch = (end - start + 15) >> 4

        def sbody(tt, _):
            j = start + tt * 16 + lane
            msk = j < end
            js = jnp.where(msk, j, 0)
            pj = plsc.load_gather(perm_v, [js])
            lf = plsc.load_gather(lflat_v, [js])
            vv = plsc.load_gather(v_v, [pj])
            plsc.store_scatter(rbuf, [lf - r0 * NPAD], vv, mask=msk)
            return 0

        lax.fori_loop(0, nch, sbody, 0)

        def rbody(r, cur):
            base = r * NPAD
            acc = jnp.full((16,), NEG, jnp.float32)
            for c in range(32):
                x = rbuf[pl.ds(base + c * 16, 16)]
                if c == 31:
                    x = jnp.where(tail_mask, x, NEG)
                acc = jnp.maximum(acc, x)
            m = lax.reduce_max(acc, axes=(0,))
            sacc = jnp.zeros((16,), jnp.float32)
            cacc = jnp.zeros((16,), jnp.float32)
            for c in range(32):
                sl = pl.ds(base + c * 16, 16)
                e = jnp.exp(rbuf[sl] - m)
                if c == 31:
                    e = jnp.where(tail_mask, e, 0.0)
                rbuf[sl] = e
                sacc = sacc + e
                cacc = cacc + e * hn_v[pl.ds(c * 16, 16)]
            s = lax.reduce_sum(sacc, axes=(0,))
            cs = lax.reduce_sum(cacc, axes=(0,))
            rcp = _vrcp(jnp.full((16,), s, jnp.float32))
            for c in range(32):
                sl = pl.ds(base + c * 16, 16)
                rbuf[sl] = rbuf[sl] * rcp
            cur = jnp.where(lane == (r & 15), cs * rcp, cur)

            @pl.when((r & 15) == 15)
            def _flush():
                cp_v[pl.ds((r >> 4) * 16, 16)] = cur
            return cur

        cur_fin = lax.fori_loop(0, RCHUNK, rbody,
                                jnp.zeros((16,), jnp.float32))
        cp_v[pl.ds(112, 16)] = cur_fin
        pltpu.sync_copy(rbuf, r_hbm.at[pl.ds(t * RB, RB)])
        pltpu.sync_copy(cp_v, cp_hbm.at[pl.ds(t * 128, 128)])


def kernel(pm25, feature, edge_attr, wind_mean, wind_std, W_ih, W_hh, b_ih,
           b_hh, W1, b1, W2, b2, Wn, bn, Wn2, bn2, edge_index):
    src = edge_index[0]
    dst = edge_index[1]
    perm = jnp.argsort(dst, stable=True).astype(jnp.int32)
    src_s = src[perm]
    dst_s = dst[perm]
    flat = src + dst * N
    winner = jnp.zeros((N * N,), jnp.int32).at[flat].max(
        jnp.arange(E, dtype=jnp.int32))
    keep_s = winner[flat][perm] == perm
    lflat = dst_s * NPAD + jnp.where(keep_s, src_s, NPAD - 4)
    counts = jnp.zeros((N,), jnp.int32).at[dst].add(1)
    off = jnp.concatenate([jnp.zeros((1,), jnp.int32),
                           jnp.cumsum(counts, dtype=jnp.int32),
                           jnp.zeros((NPAD - N - 1,), jnp.int32)])

    eaT = edge_attr.T
    featw = jnp.moveaxis(feature[:, HIST:, :, FD - 2:], -1, 0).reshape(
        2, NP * N)
    node_in = jnp.moveaxis(jnp.concatenate(
        [jnp.broadcast_to(pm25[:, 0][:, None], (B, PRED, N, 1)),
         feature[:, HIST:]], axis=-1), -1, 0).reshape(1 + FD, NP * N)

    base_gx, c1, c2, a_nodes, b_nodes, hn = pl.pallas_call(
        _prep_body,
        out_shape=(
            jax.ShapeDtypeStruct((E, 3 * H), jnp.float32),
            jax.ShapeDtypeStruct((1, E), jnp.float32),
            jax.ShapeDtypeStruct((1, E), jnp.float32),
            jax.ShapeDtypeStruct((1, NP * N), jnp.float32),
            jax.ShapeDtypeStruct((1, NP * N), jnp.float32),
            jax.ShapeDtypeStruct((1, NP * N), jnp.float32),
        ),
        in_specs=[
            pl.BlockSpec((2, E), lambda: (0, 0)),
            pl.BlockSpec((2, 3 * H), lambda: (0, 0)),
            pl.BlockSpec((1, 3 * H), lambda: (0, 0)),
            pl.BlockSpec((2, NP * N), lambda: (0, 0)),
            pl.BlockSpec((1 + FD, NP * N), lambda: (0, 0)),
            pl.BlockSpec((1 + FD, 32), lambda: (0, 0)),
            pl.BlockSpec((32, 1), lambda: (0, 0)),
            pl.BlockSpec((32, 1), lambda: (0, 0)),
            pl.BlockSpec(memory_space=pltpu.SMEM),
            pl.BlockSpec(memory_space=pltpu.SMEM),
            pl.BlockSpec(memory_space=pltpu.SMEM),
        ],
        out_specs=(
            pl.BlockSpec((E, 3 * H), lambda: (0, 0)),
            pl.BlockSpec((1, E), lambda: (0, 0)),
            pl.BlockSpec((1, E), lambda: (0, 0)),
            pl.BlockSpec((1, NP * N), lambda: (0, 0)),
            pl.BlockSpec((1, NP * N), lambda: (0, 0)),
            pl.BlockSpec((1, NP * N), lambda: (0, 0)),
        ),
    )(eaT, W_ih[:2], b_ih.reshape(1, 3 * H), featw, node_in,
      Wn, bn.reshape(32, 1), Wn2, bn2, wind_mean, wind_std)

    def _padrows(x):
        return jnp.pad(x.reshape(NP, N), ((0, 0), (0, NPAD - N))).reshape(-1)

    mesh = plsc.VectorSubcoreMesh(core_axis_name="c", subcore_axis_name="s",
                                  num_cores=2, num_subcores=16)
    sc_params = pltpu.CompilerParams(needs_layout_passes=False)
    ew_flat = pl.kernel(
        _ew_body,
        out_type=jax.ShapeDtypeStruct((PRED * B * E,), jnp.float32),
        mesh=mesh,
        compiler_params=sc_params,
        scratch_types=[
            pltpu.VMEM((NPAD,), jnp.float32),
            pltpu.VMEM((NPAD,), jnp.float32),
            pltpu.VMEM((EH,), jnp.int32),
            pltpu.VMEM((EH,), jnp.float32),
            pltpu.VMEM((EH,), jnp.float32),
            pltpu.VMEM((EH,), jnp.float32),
        ],
    )(c1.reshape(E), c2.reshape(E), _padrows(a_nodes), _padrows(b_nodes), src)

    NBLK = B * E // GBLK
    en_rep = pl.pallas_call(
        _gru_body,
        grid=(NBLK,),
        out_shape=jax.ShapeDtypeStruct((NBLK, PRED, GBLK), jnp.float32),
        in_specs=[
            pl.BlockSpec((GBLK, 3 * H), lambda k: (k % (E // GBLK), 0)),
            pl.BlockSpec((1, PRED, GBLK), lambda k: (k, 0, 0)),
            pl.BlockSpec((H, 3 * H), lambda k: (0, 0)),
            pl.BlockSpec((1, 3 * H), lambda k: (0, 0)),
            pl.BlockSpec((1, 3 * H), lambda k: (0, 0)),
            pl.BlockSpec((H, H), lambda k: (0, 0)),
            pl.BlockSpec((1, H), lambda k: (0, 0)),
            pl.BlockSpec((H, 1), lambda k: (0, 0)),
            pl.BlockSpec((1, 1), lambda k: (0, 0)),
        ],
        out_specs=pl.BlockSpec((1, PRED, GBLK), lambda k: (k, 0, 0)),
    )(base_gx, ew_flat.reshape(NBLK, PRED, GBLK), W_hh,
      b_hh.reshape(1, 3 * H), W_ih[2:3], W1, b1.reshape(1, H), W2,
      b2.reshape(1, 1))

    zeros_big = jnp.zeros((RB,), jnp.float32)
    r_out, cp_out = pl.kernel(
        _scatter_body,
        out_type=(
            jax.ShapeDtypeStruct((NTASK * RB,), jnp.float32),
            jax.ShapeDtypeStruct((NTASK * 128,), jnp.float32),
        ),
        mesh=mesh,
        compiler_params=sc_params,
        scratch_types=[
            pltpu.VMEM((E,), jnp.int32),
            pltpu.VMEM((E,), jnp.int32),
            pltpu.VMEM((NPAD,), jnp.int32),
            pltpu.VMEM((E,), jnp.float32),
            pltpu.VMEM((NPAD,), jnp.float32),
            pltpu.VMEM((RB,), jnp.float32),
            pltpu.VMEM((128,), jnp.float32),
        ],
    )(en_rep.reshape(PRED * B * E), _padrows(hn), perm, lflat, off,
      zeros_big)

    R = r_out.reshape(NP * N, NPAD)[:, :N].reshape(B, PRED, N, N)
    pm_pred = cp_out.reshape(NP, 4, 128)[:, :, :RCHUNK].reshape(
        B, PRED, N)[..., None]
    return pm_pred, R
